# Initial kernel scaffold; baseline (speedup 1.0000x reference)
import jax
import jax.numpy as jnp
from jax.experimental import pallas as pl


def kernel(x, edge_index, batch, W1, b1, W2, b2, Wc, bc):
    raise NotImplementedError("write your pallas kernel here")


# speedup vs baseline: 18.4454x; 18.4454x over previous
import functools

import jax
import jax.numpy as jnp
from jax import lax
from jax.experimental import pallas as pl
from jax.experimental.pallas import tpu as pltpu
from jax.experimental.pallas import tpu_sc as plsc

N = 10000
N_PAD = 10240
C = 128
E = 320000
G = 128

NC = 2
NS = 16
NW = NC * NS
EPT = E // NW
KB = 80
NB = EPT // KB
RPT = N_PAD // NS

_mesh = plsc.VectorSubcoreMesh(core_axis_name="c", subcore_axis_name="s")


def _deg_body(dst_hbm, out_hbm, dstv, hist):
    c = lax.axis_index("c")
    s = lax.axis_index("s")
    wid = c * NS + s
    pltpu.sync_copy(dst_hbm.at[wid], dstv)
    zeros = jnp.zeros((16,), jnp.float32)
    ones = jnp.ones((16,), jnp.float32)

    def zbody(i, carry):
        hist[pl.ds(pl.multiple_of(i * 16, 16), 16)] = zeros
        return carry

    lax.fori_loop(0, N_PAD // 16, zbody, 0)

    def body(i, carry):
        j = i // (KB // 16)
        k = i % (KB // 16)
        idx = dstv[j, pl.ds(pl.multiple_of(k * 16, 16), 16)]
        plsc.addupdate_scatter(hist, [idx], ones)
        return carry

    lax.fori_loop(0, EPT // 16, body, 0)
    pltpu.sync_copy(hist, out_hbm.at[wid])


_deg_kernel = pl.kernel(
    _deg_body,
    out_type=jax.ShapeDtypeStruct((NW, N_PAD), jnp.float32),
    mesh=_mesh,
    compiler_params=pltpu.CompilerParams(needs_layout_passes=False),
    scratch_types=[
        pltpu.VMEM((NB, KB), jnp.int32),
        pltpu.VMEM((N_PAD,), jnp.float32),
    ],
)


def _scatter_body(y_hbm, src_hbm, dst_hbm, out_hbm, srcv, dstv, rows, acc, sem):
    c = lax.axis_index("c")
    s = lax.axis_index("s")
    wid = c * NS + s
    zeros = jnp.zeros((16,), jnp.float32)

    def zbody(t, carry):
        r = t // (C // 16)
        j = t % (C // 16)
        rows[r, pl.ds(pl.multiple_of(j * 16, 16), 16)] = zeros
        return carry

    lax.fori_loop(0, KB * C // 16, zbody, 0)
    row0 = s * RPT
    for k in range(RPT // KB):
        pltpu.sync_copy(rows, acc.at[pl.ds(row0 + k * KB, KB)])
    plsc.subcore_barrier()

    pltpu.sync_copy(src_hbm.at[wid], srcv)
    pltpu.sync_copy(dst_hbm.at[wid], dstv)

    def body(j, carry):
        pltpu.async_copy(y_hbm.at[srcv.at[j]], rows, sem).wait()
        pltpu.sync_copy(rows, acc.at[dstv.at[j]], add=True)
        return carry

    lax.fori_loop(0, NB, body, 0)
    plsc.subcore_barrier()
    pltpu.sync_copy(acc.at[pl.ds(row0, RPT)], out_hbm.at[c, pl.ds(row0, RPT)])


_scatter_kernel = pl.kernel(
    _scatter_body,
    out_type=jax.ShapeDtypeStruct((NC, N_PAD, C), jnp.float32),
    mesh=_mesh,
    scratch_types=[
        pltpu.VMEM((NB, KB), jnp.int32),
        pltpu.VMEM((NB, KB), jnp.int32),
        pltpu.VMEM((KB, C), jnp.float32),
        pltpu.VMEM_SHARED((N_PAD, C), jnp.float32),
        pltpu.SemaphoreType.DMA,
    ],
)


BLK = 256
GRID = N_PAD // BLK


def _tc_in_body(deg_ref, x_ref, w_ref, y_ref, dinv_ref):
    ones = jnp.ones((NW, 1), jnp.float32)
    deg = lax.dot_general(deg_ref[...], ones, (((0,), (0,)), ((), ())),
                          preferred_element_type=jnp.float32) + 1.0
    dinv = lax.rsqrt(deg)
    xw = jnp.dot(x_ref[...], w_ref[...], preferred_element_type=jnp.float32)
    y_ref[...] = xw * dinv
    dinv_ref[...] = dinv


_tc_in = pl.pallas_call(
    _tc_in_body,
    grid=(GRID,),
    in_specs=[
        pl.BlockSpec((NW, BLK), lambda i: (0, i)),
        pl.BlockSpec((BLK, C), lambda i: (i, 0)),
        pl.BlockSpec((C, C), lambda i: (0, 0)),
    ],
    out_specs=[
        pl.BlockSpec((BLK, C), lambda i: (i, 0)),
        pl.BlockSpec((BLK, 1), lambda i: (i, 0)),
    ],
    out_shape=[
        jax.ShapeDtypeStruct((N_PAD, C), jnp.float32),
        jax.ShapeDtypeStruct((N_PAD, 1), jnp.float32),
    ],
)


def _tc_mid_body(s_ref, y_ref, dinv_ref, b_ref, w_ref, y2_ref):
    sm = s_ref[0] + s_ref[1]
    dinv = dinv_ref[...]
    h = jnp.maximum((sm + y_ref[...]) * dinv + b_ref[...], 0.0)
    y2_ref[...] = jnp.dot(h, w_ref[...], preferred_element_type=jnp.float32) * dinv


_tc_mid = pl.pallas_call(
    _tc_mid_body,
    grid=(GRID,),
    in_specs=[
        pl.BlockSpec((NC, BLK, C), lambda i: (0, i, 0)),
        pl.BlockSpec((BLK, C), lambda i: (i, 0)),
        pl.BlockSpec((BLK, 1), lambda i: (i, 0)),
        pl.BlockSpec((C,), lambda i: (0,)),
        pl.BlockSpec((C, C), lambda i: (0, 0)),
    ],
    out_specs=pl.BlockSpec((BLK, C), lambda i: (i, 0)),
    out_shape=jax.ShapeDtypeStruct((N_PAD, C), jnp.float32),
)


def _tc_fin_body(s_ref, y2_ref, dinv_ref, b_ref, batch_ref, wc_ref, bc_ref,
                 out_ref, pooled, cnt):
    i = pl.program_id(0)

    @pl.when(i == 0)
    def _():
        pooled[...] = jnp.zeros_like(pooled)
        cnt[...] = jnp.zeros_like(cnt)

    sm = s_ref[0] + s_ref[1]
    o2 = (sm + y2_ref[...]) * dinv_ref[...] + b_ref[...]
    onehot = (batch_ref[...] == lax.broadcasted_iota(jnp.int32, (BLK, G), 1)
              ).astype(jnp.float32)
    pooled[...] += lax.dot_general(onehot, o2, (((0,), (0,)), ((), ())),
                                   preferred_element_type=jnp.float32)
    cnt[...] += lax.dot_general(onehot, jnp.ones((BLK, 1), jnp.float32),
                                (((0,), (0,)), ((), ())),
                                preferred_element_type=jnp.float32)

    @pl.when(i == GRID - 1)
    def _():
        g = pooled[...] / jnp.maximum(cnt[...], 1.0)
        out_ref[...] = jnp.dot(g, wc_ref[...],
                               preferred_element_type=jnp.float32) + bc_ref[...]


_tc_fin = pl.pallas_call(
    _tc_fin_body,
    grid=(GRID,),
    in_specs=[
        pl.BlockSpec((NC, BLK, C), lambda i: (0, i, 0)),
        pl.BlockSpec((BLK, C), lambda i: (i, 0)),
        pl.BlockSpec((BLK, 1), lambda i: (i, 0)),
        pl.BlockSpec((C,), lambda i: (0,)),
        pl.BlockSpec((BLK, 1), lambda i: (i, 0)),
        pl.BlockSpec((C, 1), lambda i: (0, 0)),
        pl.BlockSpec((1, 1), lambda i: (0, 0)),
    ],
    out_specs=pl.BlockSpec((G, 1), lambda i: (0, 0)),
    out_shape=jax.ShapeDtypeStruct((G, 1), jnp.float32),
    scratch_shapes=[
        pltpu.VMEM((G, C), jnp.float32),
        pltpu.VMEM((G, 1), jnp.float32),
    ],
)


def kernel(x, edge_index, batch, W1, b1, W2, b2, Wc, bc):
    src = edge_index[0].astype(jnp.int32).reshape(NW, NB, KB)
    dst = edge_index[1].astype(jnp.int32).reshape(NW, NB, KB)
    x_p = jnp.zeros((N_PAD, C), jnp.float32).at[:N].set(x)
    batch_p = jnp.full((N_PAD, 1), G, jnp.int32).at[:N, 0].set(
        batch.astype(jnp.int32))
    bc2 = bc.reshape(1, 1)

    degp = _deg_kernel(dst)
    y1, dinv = _tc_in(degp, x_p, W1)
    s1 = _scatter_kernel(y1, src, dst)
    y2 = _tc_mid(s1, y1, dinv, b1, W2)
    s2 = _scatter_kernel(y2, src, dst)
    out = _tc_fin(s2, y2, dinv, b2, batch_p, Wc, bc2)
    return out


# speedup vs baseline: 25.9614x; 1.4075x over previous
import functools

import jax
import jax.numpy as jnp
from jax import lax
from jax.experimental import pallas as pl
from jax.experimental.pallas import tpu as pltpu
from jax.experimental.pallas import tpu_sc as plsc

N = 10000
N_PAD = 10240
C = 128
E = 320000
G = 128

NC = 2
NS = 16
NW = NC * NS
EPT = E // NW
KB = 80
NB = EPT // KB
RPT = N_PAD // NS

_mesh = plsc.VectorSubcoreMesh(core_axis_name="c", subcore_axis_name="s")


def _deg_body(dst_hbm, out_hbm, dstv, hist):
    c = lax.axis_index("c")
    s = lax.axis_index("s")
    wid = c * NS + s
    pltpu.sync_copy(dst_hbm.at[wid], dstv)
    zeros = jnp.zeros((16,), jnp.float32)
    ones = jnp.ones((16,), jnp.float32)

    def zbody(i, carry):
        hist[pl.ds(pl.multiple_of(i * 16, 16), 16)] = zeros
        return carry

    lax.fori_loop(0, N_PAD // 16, zbody, 0)

    def body(i, carry):
        idx = dstv[i]
        plsc.addupdate_scatter(hist, [idx], ones)
        return carry

    lax.fori_loop(0, EPT // 16, body, 0)
    pltpu.sync_copy(hist, out_hbm.at[wid])


_deg_kernel = pl.kernel(
    _deg_body,
    out_type=jax.ShapeDtypeStruct((NW, N_PAD), jnp.float32),
    mesh=_mesh,
    compiler_params=pltpu.CompilerParams(needs_layout_passes=False),
    scratch_types=[
        pltpu.VMEM((EPT // 16, 16), jnp.int32),
        pltpu.VMEM((N_PAD,), jnp.float32),
    ],
)


NBUF = 2
NT = (NB - 2 * NBUF) // NBUF + 1
REM = NB - (NT + 1) * NBUF


def _gidx(srcv, j):
    return srcv.at[pl.ds(pl.multiple_of(j * KB, 8), KB)]


def _scatter_body(y_hbm, src_hbm, dst_hbm, out_hbm, srcv, dstv, rows, acc, sem):
    c = lax.axis_index("c")
    s = lax.axis_index("s")
    wid = c * NS + s
    zeros = jnp.zeros((16,), jnp.float32)

    pltpu.sync_copy(src_hbm.at[wid], srcv)
    pltpu.sync_copy(dst_hbm.at[wid], dstv)

    def zbody(t, carry):
        r = t // (C // 16)
        j = t % (C // 16)
        rows[0, r, pl.ds(pl.multiple_of(j * 16, 16), 16)] = zeros
        return carry

    lax.fori_loop(0, KB * C // 16, zbody, 0)
    row0 = s * RPT
    for k in range(RPT // KB):
        pltpu.sync_copy(rows.at[0], acc.at[pl.ds(row0 + k * KB, KB)])

    for b in range(NBUF):
        pltpu.async_copy(y_hbm.at[_gidx(srcv, b)], rows.at[b], sem.at[b])
    plsc.subcore_barrier()

    def body(t, carry):
        for b in range(NBUF):
            j = t * NBUF + b
            pltpu.make_async_copy(y_hbm.at[_gidx(srcv, j)], rows.at[b],
                                  sem.at[b]).wait()
            pltpu.sync_copy(rows.at[b], acc.at[dstv.at[j]], add=True)
            pltpu.async_copy(y_hbm.at[_gidx(srcv, j + NBUF)], rows.at[b],
                             sem.at[b])
        return carry

    lax.fori_loop(0, NT, body, 0)
    for b in range(NBUF):
        j = NT * NBUF + b
        pltpu.make_async_copy(y_hbm.at[_gidx(srcv, j)], rows.at[b],
                              sem.at[b]).wait()
        pltpu.sync_copy(rows.at[b], acc.at[dstv.at[j]], add=True)
        if j + NBUF < NB:
            pltpu.async_copy(y_hbm.at[_gidx(srcv, j + NBUF)], rows.at[b],
                             sem.at[b])
    for b in range(REM):
        j = NT * NBUF + NBUF + b
        pltpu.make_async_copy(y_hbm.at[_gidx(srcv, j)], rows.at[b],
                              sem.at[b]).wait()
        pltpu.sync_copy(rows.at[b], acc.at[dstv.at[j]], add=True)
    plsc.subcore_barrier()
    pltpu.sync_copy(acc.at[pl.ds(row0, RPT)], out_hbm.at[c, pl.ds(row0, RPT)])


_scatter_kernel = pl.kernel(
    _scatter_body,
    out_type=jax.ShapeDtypeStruct((NC, N_PAD, C), jnp.float32),
    mesh=_mesh,
    scratch_types=[
        pltpu.VMEM((EPT,), jnp.int32),
        pltpu.VMEM((NB, KB), jnp.int32),
        pltpu.VMEM((NBUF, KB, C), jnp.float32),
        pltpu.VMEM_SHARED((N_PAD, C), jnp.float32),
        pltpu.SemaphoreType.DMA((NBUF,)),
    ],
)


BLK = 256
GRID = N_PAD // BLK


_HI = lax.Precision.HIGHEST


def _tc_in_body(deg_ref, x_ref, w_ref, y_ref, dinv_ref):
    ones = jnp.ones((NW, 1), jnp.float32)
    deg = lax.dot_general(deg_ref[...], ones, (((0,), (0,)), ((), ())),
                          preferred_element_type=jnp.float32,
                          precision=_HI) + 1.0
    r = lax.rsqrt(deg)
    dinv = r * (1.5 - 0.5 * deg * r * r)
    xw = jnp.dot(x_ref[...], w_ref[...], preferred_element_type=jnp.float32,
                 precision=_HI)
    y_ref[...] = xw * dinv
    dinv_ref[...] = dinv


_tc_in = pl.pallas_call(
    _tc_in_body,
    grid=(GRID,),
    in_specs=[
        pl.BlockSpec((NW, BLK), lambda i: (0, i)),
        pl.BlockSpec((BLK, C), lambda i: (i, 0)),
        pl.BlockSpec((C, C), lambda i: (0, 0)),
    ],
    out_specs=[
        pl.BlockSpec((BLK, C), lambda i: (i, 0)),
        pl.BlockSpec((BLK, 1), lambda i: (i, 0)),
    ],
    out_shape=[
        jax.ShapeDtypeStruct((N_PAD, C), jnp.float32),
        jax.ShapeDtypeStruct((N_PAD, 1), jnp.float32),
    ],
)


def _tc_mid_body(s_ref, y_ref, dinv_ref, b_ref, w_ref, y2_ref):
    sm = s_ref[0] + s_ref[1]
    dinv = dinv_ref[...]
    h = jnp.maximum((sm + y_ref[...]) * dinv + b_ref[...], 0.0)
    y2_ref[...] = jnp.dot(h, w_ref[...], preferred_element_type=jnp.float32,
                          precision=_HI) * dinv


_tc_mid = pl.pallas_call(
    _tc_mid_body,
    grid=(GRID,),
    in_specs=[
        pl.BlockSpec((NC, BLK, C), lambda i: (0, i, 0)),
        pl.BlockSpec((BLK, C), lambda i: (i, 0)),
        pl.BlockSpec((BLK, 1), lambda i: (i, 0)),
        pl.BlockSpec((C,), lambda i: (0,)),
        pl.BlockSpec((C, C), lambda i: (0, 0)),
    ],
    out_specs=pl.BlockSpec((BLK, C), lambda i: (i, 0)),
    out_shape=jax.ShapeDtypeStruct((N_PAD, C), jnp.float32),
)


def _tc_fin_body(s_ref, y2_ref, dinv_ref, b_ref, batch_ref, wc_ref, bc_ref,
                 out_ref, pooled, cnt):
    i = pl.program_id(0)

    @pl.when(i == 0)
    def _():
        pooled[...] = jnp.zeros_like(pooled)
        cnt[...] = jnp.zeros_like(cnt)

    sm = s_ref[0] + s_ref[1]
    o2 = (sm + y2_ref[...]) * dinv_ref[...] + b_ref[...]
    onehot = (batch_ref[...] == lax.broadcasted_iota(jnp.int32, (BLK, G), 1)
              ).astype(jnp.float32)
    pooled[...] += lax.dot_general(onehot, o2, (((0,), (0,)), ((), ())),
                                   preferred_element_type=jnp.float32,
                                   precision=_HI)
    cnt[...] += lax.dot_general(onehot, jnp.ones((BLK, 1), jnp.float32),
                                (((0,), (0,)), ((), ())),
                                preferred_element_type=jnp.float32,
                                precision=_HI)

    @pl.when(i == GRID - 1)
    def _():
        g = pooled[...] / jnp.maximum(cnt[...], 1.0)
        out_ref[...] = jnp.dot(g, wc_ref[...], preferred_element_type=jnp.float32,
                               precision=_HI) + bc_ref[...]


_tc_fin = pl.pallas_call(
    _tc_fin_body,
    grid=(GRID,),
    in_specs=[
        pl.BlockSpec((NC, BLK, C), lambda i: (0, i, 0)),
        pl.BlockSpec((BLK, C), lambda i: (i, 0)),
        pl.BlockSpec((BLK, 1), lambda i: (i, 0)),
        pl.BlockSpec((C,), lambda i: (0,)),
        pl.BlockSpec((BLK, 1), lambda i: (i, 0)),
        pl.BlockSpec((C, 1), lambda i: (0, 0)),
        pl.BlockSpec((1, 1), lambda i: (0, 0)),
    ],
    out_specs=pl.BlockSpec((G, 1), lambda i: (0, 0)),
    out_shape=jax.ShapeDtypeStruct((G, 1), jnp.float32),
    scratch_shapes=[
        pltpu.VMEM((G, C), jnp.float32),
        pltpu.VMEM((G, 1), jnp.float32),
    ],
)


def kernel(x, edge_index, batch, W1, b1, W2, b2, Wc, bc):
    src = edge_index[0].astype(jnp.int32).reshape(NW, EPT)
    dst = edge_index[1].astype(jnp.int32).reshape(NW, NB, KB)
    dst16 = edge_index[1].astype(jnp.int32).reshape(NW, EPT // 16, 16)
    x_p = jnp.zeros((N_PAD, C), jnp.float32).at[:N].set(x)
    batch_p = jnp.full((N_PAD, 1), G, jnp.int32).at[:N, 0].set(
        batch.astype(jnp.int32))
    bc2 = bc.reshape(1, 1)

    degp = _deg_kernel(dst16)
    y1, dinv = _tc_in(degp, x_p, W1)
    s1 = _scatter_kernel(y1, src, dst)
    y2 = _tc_mid(s1, y1, dinv, b1, W2)
    s2 = _scatter_kernel(y2, src, dst)
    out = _tc_fin(s2, y2, dinv, b2, batch_p, Wc, bc2)
    return out


# speedup vs baseline: 26.6402x; 1.0261x over previous
import functools

import jax
import jax.numpy as jnp
from jax import lax
from jax.experimental import pallas as pl
from jax.experimental.pallas import tpu as pltpu
from jax.experimental.pallas import tpu_sc as plsc

N = 10000
N_PAD = 10240
C = 128
E = 320000
G = 128

NC = 2
NS = 16
NW = NC * NS
EPT = E // NW
KB = 80
NB = EPT // KB
RPT = N_PAD // NS

_mesh = plsc.VectorSubcoreMesh(core_axis_name="c", subcore_axis_name="s")


def _deg_body(dst_hbm, out_hbm, dstv, hist):
    c = lax.axis_index("c")
    s = lax.axis_index("s")
    wid = c * NS + s
    pltpu.sync_copy(dst_hbm.at[wid], dstv)
    zeros = jnp.zeros((16,), jnp.float32)
    ones = jnp.ones((16,), jnp.float32)

    def zbody(i, carry):
        hist[pl.ds(pl.multiple_of(i * 16, 16), 16)] = zeros
        return carry

    lax.fori_loop(0, N_PAD // 16, zbody, 0)

    def body(i, carry):
        idx = dstv[i]
        plsc.addupdate_scatter(hist, [idx], ones)
        return carry

    lax.fori_loop(0, EPT // 16, body, 0)
    pltpu.sync_copy(hist, out_hbm.at[wid])


_deg_kernel = pl.kernel(
    _deg_body,
    out_type=jax.ShapeDtypeStruct((NW, N_PAD), jnp.float32),
    mesh=_mesh,
    compiler_params=pltpu.CompilerParams(needs_layout_passes=False),
    scratch_types=[
        pltpu.VMEM((EPT // 16, 16), jnp.int32),
        pltpu.VMEM((N_PAD,), jnp.float32),
    ],
)


NBUF = 2
NT = (NB - 2 * NBUF) // NBUF + 1
REM = NB - (NT + 1) * NBUF


def _gidx(srcv, j):
    return srcv.at[pl.ds(pl.multiple_of(j * KB, 8), KB)]


def _scatter_body(y_hbm, src_hbm, dst_hbm, out_hbm, srcv, dstv, rows, acc, sem):
    c = lax.axis_index("c")
    s = lax.axis_index("s")
    wid = c * NS + s
    zeros = jnp.zeros((16,), jnp.float32)

    pltpu.sync_copy(src_hbm.at[wid], srcv)
    pltpu.sync_copy(dst_hbm.at[wid], dstv)

    def zbody(t, carry):
        r = t // (C // 16)
        j = t % (C // 16)
        rows[0, r, pl.ds(pl.multiple_of(j * 16, 16), 16)] = zeros
        return carry

    lax.fori_loop(0, KB * C // 16, zbody, 0)
    row0 = s * RPT
    for k in range(RPT // KB):
        pltpu.sync_copy(rows.at[0], acc.at[pl.ds(row0 + k * KB, KB)])

    for b in range(NBUF):
        pltpu.async_copy(y_hbm.at[_gidx(srcv, b)], rows.at[b], sem.at[b])
    plsc.subcore_barrier()

    def body(t, carry):
        for b in range(NBUF):
            j = t * NBUF + b
            pltpu.make_async_copy(y_hbm.at[_gidx(srcv, j)], rows.at[b],
                                  sem.at[b]).wait()
            pltpu.sync_copy(rows.at[b], acc.at[dstv.at[j]], add=True)
            pltpu.async_copy(y_hbm.at[_gidx(srcv, j + NBUF)], rows.at[b],
                             sem.at[b])
        return carry

    lax.fori_loop(0, NT, body, 0)
    for b in range(NBUF):
        j = NT * NBUF + b
        pltpu.make_async_copy(y_hbm.at[_gidx(srcv, j)], rows.at[b],
                              sem.at[b]).wait()
        pltpu.sync_copy(rows.at[b], acc.at[dstv.at[j]], add=True)
        if j + NBUF < NB:
            pltpu.async_copy(y_hbm.at[_gidx(srcv, j + NBUF)], rows.at[b],
                             sem.at[b])
    for b in range(REM):
        j = NT * NBUF + NBUF + b
        pltpu.make_async_copy(y_hbm.at[_gidx(srcv, j)], rows.at[b],
                              sem.at[b]).wait()
        pltpu.sync_copy(rows.at[b], acc.at[dstv.at[j]], add=True)
    plsc.subcore_barrier()
    pltpu.sync_copy(acc.at[pl.ds(row0, RPT)], out_hbm.at[c, pl.ds(row0, RPT)])


_scatter_kernel = pl.kernel(
    _scatter_body,
    out_type=jax.ShapeDtypeStruct((NC, N_PAD, C), jnp.float32),
    mesh=_mesh,
    scratch_types=[
        pltpu.VMEM((EPT,), jnp.int32),
        pltpu.VMEM((NB, KB), jnp.int32),
        pltpu.VMEM((NBUF, KB, C), jnp.float32),
        pltpu.VMEM_SHARED((N_PAD, C), jnp.float32),
        pltpu.SemaphoreType.DMA((NBUF,)),
    ],
)


BLK = 256
GRID = N_PAD // BLK


def _tc_in_body(deg_ref, x_ref, w_ref, y_ref, dinv_ref):
    ones = jnp.ones((NW, 1), jnp.float32)
    deg = lax.dot_general(deg_ref[...], ones, (((0,), (0,)), ((), ())),
                          preferred_element_type=jnp.float32) + 1.0
    dinv = lax.rsqrt(deg)
    xw = jnp.dot(x_ref[...], w_ref[...], preferred_element_type=jnp.float32)
    y_ref[...] = xw * dinv
    dinv_ref[...] = dinv


_tc_in = pl.pallas_call(
    _tc_in_body,
    grid=(GRID,),
    in_specs=[
        pl.BlockSpec((NW, BLK), lambda i: (0, i)),
        pl.BlockSpec((BLK, C), lambda i: (i, 0)),
        pl.BlockSpec((C, C), lambda i: (0, 0)),
    ],
    out_specs=[
        pl.BlockSpec((BLK, C), lambda i: (i, 0)),
        pl.BlockSpec((BLK, 1), lambda i: (i, 0)),
    ],
    out_shape=[
        jax.ShapeDtypeStruct((N_PAD, C), jnp.float32),
        jax.ShapeDtypeStruct((N_PAD, 1), jnp.float32),
    ],
)


def _tc_mid_body(s_ref, y_ref, dinv_ref, b_ref, w_ref, y2_ref):
    sm = s_ref[0] + s_ref[1]
    dinv = dinv_ref[...]
    h = jnp.maximum((sm + y_ref[...]) * dinv + b_ref[...], 0.0)
    y2_ref[...] = jnp.dot(h, w_ref[...], preferred_element_type=jnp.float32) * dinv


_tc_mid = pl.pallas_call(
    _tc_mid_body,
    grid=(GRID,),
    in_specs=[
        pl.BlockSpec((NC, BLK, C), lambda i: (0, i, 0)),
        pl.BlockSpec((BLK, C), lambda i: (i, 0)),
        pl.BlockSpec((BLK, 1), lambda i: (i, 0)),
        pl.BlockSpec((C,), lambda i: (0,)),
        pl.BlockSpec((C, C), lambda i: (0, 0)),
    ],
    out_specs=pl.BlockSpec((BLK, C), lambda i: (i, 0)),
    out_shape=jax.ShapeDtypeStruct((N_PAD, C), jnp.float32),
)


def _tc_fin_body(s_ref, y2_ref, dinv_ref, b_ref, batch_ref, wc_ref, bc_ref,
                 out_ref, pooled, cnt):
    i = pl.program_id(0)

    @pl.when(i == 0)
    def _():
        pooled[...] = jnp.zeros_like(pooled)
        cnt[...] = jnp.zeros_like(cnt)

    sm = s_ref[0] + s_ref[1]
    o2 = (sm + y2_ref[...]) * dinv_ref[...] + b_ref[...]
    onehot = (batch_ref[...] == lax.broadcasted_iota(jnp.int32, (BLK, G), 1)
              ).astype(jnp.float32)
    pooled[...] += lax.dot_general(onehot, o2, (((0,), (0,)), ((), ())),
                                   preferred_element_type=jnp.float32)
    cnt[...] += lax.dot_general(onehot, jnp.ones((BLK, 1), jnp.float32),
                                (((0,), (0,)), ((), ())),
                                preferred_element_type=jnp.float32)

    @pl.when(i == GRID - 1)
    def _():
        g = pooled[...] / jnp.maximum(cnt[...], 1.0)
        out_ref[...] = jnp.dot(g, wc_ref[...], preferred_element_type=jnp.float32) + bc_ref[...]


_tc_fin = pl.pallas_call(
    _tc_fin_body,
    grid=(GRID,),
    in_specs=[
        pl.BlockSpec((NC, BLK, C), lambda i: (0, i, 0)),
        pl.BlockSpec((BLK, C), lambda i: (i, 0)),
        pl.BlockSpec((BLK, 1), lambda i: (i, 0)),
        pl.BlockSpec((C,), lambda i: (0,)),
        pl.BlockSpec((BLK, 1), lambda i: (i, 0)),
        pl.BlockSpec((C, 1), lambda i: (0, 0)),
        pl.BlockSpec((1, 1), lambda i: (0, 0)),
    ],
    out_specs=pl.BlockSpec((G, 1), lambda i: (0, 0)),
    out_shape=jax.ShapeDtypeStruct((G, 1), jnp.float32),
    scratch_shapes=[
        pltpu.VMEM((G, C), jnp.float32),
        pltpu.VMEM((G, 1), jnp.float32),
    ],
)


def kernel(x, edge_index, batch, W1, b1, W2, b2, Wc, bc):
    src = edge_index[0].astype(jnp.int32).reshape(NW, EPT)
    dst = edge_index[1].astype(jnp.int32).reshape(NW, NB, KB)
    dst16 = edge_index[1].astype(jnp.int32).reshape(NW, EPT // 16, 16)
    x_p = jnp.zeros((N_PAD, C), jnp.float32).at[:N].set(x)
    batch_p = jnp.full((N_PAD, 1), G, jnp.int32).at[:N, 0].set(
        batch.astype(jnp.int32))
    bc2 = bc.reshape(1, 1)

    degp = _deg_kernel(dst16)
    y1, dinv = _tc_in(degp, x_p, W1)
    s1 = _scatter_kernel(y1, src, dst)
    y2 = _tc_mid(s1, y1, dinv, b1, W2)
    s2 = _scatter_kernel(y2, src, dst)
    out = _tc_fin(s2, y2, dinv, b2, batch_p, Wc, bc2)
    return out
